# baseline (speedup 1.0000x reference)
import functools

import numpy as np
import jax
import jax.numpy as jnp
from jax import lax
from jax.experimental import pallas as pl
from jax.experimental.pallas import tpu as pltpu
from jax.experimental.pallas import tpu_sc as plsc

N_EXPERTS = 64
EXPANSION = 8
CAP = 1152

try:
    _SC_INFO = plsc.get_sparse_core_info()
    _NC, _NS = _SC_INFO.num_cores, _SC_INFO.num_subcores
except Exception:
    _NC, _NS = 2, 16
_NW = _NC * _NS
_EPW = N_EXPERTS // _NW

_INT_MIN = np.int32(-2147483648)


def _router_body(x_ref, rw_ref, allwt_ref, keys_ref, xb_ref):
    lgt = lax.dot_general(rw_ref[...].astype(jnp.bfloat16),
                          x_ref[...].astype(jnp.bfloat16),
                          (((1,), (1,)), ((), ())),
                          preferred_element_type=jnp.float32)
    m = jnp.max(lgt, axis=0, keepdims=True)
    e = jnp.exp(lgt - m)
    allwt_ref[...] = e / jnp.sum(e, axis=0, keepdims=True)
    u = lax.bitcast_convert_type(lgt, jnp.uint32)
    neg = u >= jnp.uint32(0x80000000)
    up = jnp.where(neg, ~u, u | jnp.uint32(0x80000000))
    keys_ref[...] = lax.bitcast_convert_type(up ^ jnp.uint32(0x80000000),
                                             jnp.int32)
    xb = x_ref[...].astype(jnp.bfloat16)
    c = xb.shape[1]
    a = lax.bitcast_convert_type(xb[:, : c // 2], jnp.uint16).astype(jnp.uint32)
    b = lax.bitcast_convert_type(xb[:, c // 2 :], jnp.uint16).astype(jnp.uint32)
    xb_ref[...] = lax.bitcast_convert_type((a << 16) | b, jnp.int32)


def _router(x_flat, router_W):
    n, c = x_flat.shape
    blk = 1024
    return pl.pallas_call(
        _router_body,
        grid=(n // blk,),
        in_specs=[
            pl.BlockSpec((blk, c), lambda i: (i, 0)),
            pl.BlockSpec((N_EXPERTS, c), lambda i: (0, 0)),
        ],
        out_specs=[
            pl.BlockSpec((N_EXPERTS, blk), lambda i: (0, i)),
            pl.BlockSpec((N_EXPERTS, blk), lambda i: (0, i)),
            pl.BlockSpec((blk, c // 2), lambda i: (i, 0)),
        ],
        out_shape=[
            jax.ShapeDtypeStruct((N_EXPERTS, n), jnp.float32),
            jax.ShapeDtypeStruct((N_EXPERTS, n), jnp.int32),
            jax.ShapeDtypeStruct((n, c // 2), jnp.int32),
        ],
    )(x_flat, router_W)


def _thresh_body(k, keys_ref, t_ref):
    keys = keys_ref[...]
    t = jnp.full((N_EXPERTS, 1), _INT_MIN, jnp.int32)
    for b in range(31, -1, -1):
        cand = t + np.array(1 << b, dtype=np.uint32).view(np.int32)
        cnt = jnp.sum((keys >= cand).astype(jnp.int32), axis=1, keepdims=True)
        t = jnp.where(cnt >= k, cand, t)
    t_ref[...] = jnp.broadcast_to(t, (N_EXPERTS, 128))


def _threshold(keys, k):
    n = keys.shape[1]
    return pl.pallas_call(
        functools.partial(_thresh_body, k),
        in_specs=[pl.BlockSpec((N_EXPERTS, n), lambda: (0, 0))],
        out_specs=pl.BlockSpec((N_EXPERTS, 128), lambda: (0, 0)),
        out_shape=jax.ShapeDtypeStruct((N_EXPERTS, 128), jnp.int32),
    )(keys)


def _make_sc_compact(n):
    n_vregs = n // 16
    mesh = plsc.VectorSubcoreMesh(core_axis_name="c", subcore_axis_name="s")

    @functools.partial(
        pl.kernel,
        mesh=mesh,
        compiler_params=pltpu.CompilerParams(needs_layout_passes=False),
        out_type=(
            jax.ShapeDtypeStruct((N_EXPERTS * CAP,), jnp.int32),
            jax.ShapeDtypeStruct((N_EXPERTS * CAP,), jnp.int32),
        ),
        scratch_types=[
            pltpu.VMEM((n,), jnp.int32),
            pltpu.VMEM((16,), jnp.int32),
            pltpu.VMEM((CAP,), jnp.int32),
            pltpu.VMEM((CAP,), jnp.int32),
        ],
    )
    def compact_k(keys_hbm, t_hbm, keysc_hbm, idxc_hbm, keys_v, t_v, kc_v, ic_v):
        wid = lax.axis_index("s") * _NC + lax.axis_index("c")
        for t in range(_EPW):
            e = wid * _EPW + t
            pltpu.sync_copy(keys_hbm.at[pl.ds(e * n, n)], keys_v)
            pltpu.sync_copy(t_hbm.at[pl.ds(e * 16, 16)], t_v)
            tvec = t_v[...]

            def fill(j, carry):
                kc_v[pl.ds(j * 16, 16)] = jnp.full((16,), _INT_MIN, jnp.int32)
                ic_v[pl.ds(j * 16, 16)] = jnp.full((16,), 0x7FFFFFFF, jnp.int32)
                return carry

            lax.fori_loop(0, CAP // 16, fill, 0)

            def step(i, cnt):
                kv = keys_v[pl.ds(i * 16, 16)]
                m = (kv >= tvec) & (cnt < CAP - 16)
                iv = lax.iota(jnp.int32, 16) + i * 16
                cs = plsc.cumsum(m.astype(jnp.int32))
                dest = cnt + cs - 1
                plsc.store_scatter(kc_v, [dest], kv, mask=m)
                plsc.store_scatter(ic_v, [dest], iv, mask=m)
                return cnt + jnp.max(cs)

            lax.fori_loop(0, n_vregs, step, jnp.int32(0))
            pltpu.sync_copy(kc_v, keysc_hbm.at[pl.ds(e * CAP, CAP)])
            pltpu.sync_copy(ic_v, idxc_hbm.at[pl.ds(e * CAP, CAP)])

    return compact_k


def _rank_body(keys_ref, rank_ref):
    keys = keys_ref[0, 0]
    kc = keys.reshape(CAP, 1)
    kr = keys.reshape(1, CAP)
    row = lax.broadcasted_iota(jnp.int32, (CAP, CAP), 0)
    col = lax.broadcasted_iota(jnp.int32, (CAP, CAP), 1)
    a = (kr > kc) | ((kr == kc) & (col < row))
    rank_ref[0, 0] = jnp.sum(a.astype(jnp.int32), axis=1)


def _ranks(keysc):
    return pl.pallas_call(
        _rank_body,
        grid=(N_EXPERTS,),
        in_specs=[pl.BlockSpec((1, 1, CAP), lambda i: (i, 0, 0))],
        out_specs=pl.BlockSpec((1, 1, CAP), lambda i: (i, 0, 0)),
        out_shape=jax.ShapeDtypeStruct((N_EXPERTS, 1, CAP), jnp.int32),
    )(keysc.reshape(N_EXPERTS, 1, CAP))


def _make_sc_place_gather(n, k, cw, e_lo, nexp):
    ch = 64
    nch = k // ch
    mesh = plsc.VectorSubcoreMesh(core_axis_name="c", subcore_axis_name="s")

    @functools.partial(
        pl.kernel,
        mesh=mesh,
        compiler_params=pltpu.CompilerParams(needs_layout_passes=False),
        out_type=(
            jax.ShapeDtypeStruct((nexp * k,), jnp.int32),
            jax.ShapeDtypeStruct((nexp * k,), jnp.float32),
            jax.ShapeDtypeStruct((_NW * n,), jnp.float32),
            jax.ShapeDtypeStruct((nexp * k, cw), jnp.int32),
        ),
        scratch_types=[
            pltpu.VMEM((CAP,), jnp.int32),
            pltpu.VMEM((CAP,), jnp.int32),
            pltpu.VMEM((k,), jnp.int32),
            pltpu.VMEM((n,), jnp.float32),
            pltpu.VMEM((k,), jnp.float32),
            pltpu.VMEM((n,), jnp.float32),
            pltpu.VMEM((64, cw), jnp.int32),
            pltpu.VMEM((64, cw), jnp.int32),
            pltpu.SemaphoreType.DMA,
            pltpu.SemaphoreType.DMA,
            pltpu.SemaphoreType.DMA,
        ],
    )
    def place_k(idxc_hbm, rank_hbm, allwt_hbm, zi_hbm, zf_hbm, xb_hbm,
                lidx_hbm, wflat_hbm, fpart_hbm, xg_hbm,
                ic_v, rk_v, tk_v, aw_v, w_v, hist_v, rb0, rb1, gsem, os0, os1):
        wid = lax.axis_index("s") * _NC + lax.axis_index("c")
        pltpu.sync_copy(zf_hbm, hist_v)
        ones = jnp.full((16,), 1.0, jnp.float32)
        rbufs = (rb0, rb1)
        osems = (os0, os1)
        epw = nexp // _NW
        for t in range(epw):
            le = wid * epw + t
            e = e_lo + le
            pltpu.sync_copy(idxc_hbm.at[pl.ds(e * CAP, CAP)], ic_v)
            pltpu.sync_copy(rank_hbm.at[pl.ds(e * CAP, CAP)], rk_v)
            pltpu.sync_copy(zi_hbm.at[pl.ds(0, k)], tk_v)

            def place(j, carry):
                r = rk_v[pl.ds(j * 16, 16)]
                iv = ic_v[pl.ds(j * 16, 16)]
                m = r < k
                plsc.store_scatter(tk_v, [r], iv, mask=m)
                return carry

            lax.fori_loop(0, CAP // 16, place, 0)
            pltpu.sync_copy(tk_v, lidx_hbm.at[pl.ds(le * k, k)])
            pltpu.sync_copy(allwt_hbm.at[pl.ds(e * n, n)], aw_v)

            def wgather(j, carry):
                tok = tk_v[pl.ds(j * 16, 16)]
                w_v[pl.ds(j * 16, 16)] = plsc.load_gather(aw_v, [tok])
                plsc.addupdate_scatter(hist_v, [tok], ones)
                return carry

            lax.fori_loop(0, k // 16, wgather, 0)
            pltpu.sync_copy(w_v, wflat_hbm.at[pl.ds(le * k, k)])

            ch = 64
            out_cps = [None, None]
            for i in range(k // ch):
                b = i & 1
                if out_cps[b] is not None:
                    out_cps[b].wait()
                pltpu.async_copy(xb_hbm.at[tk_v.at[pl.ds(i * ch, ch)]],
                                 rbufs[b], gsem).wait()
                out_cps[b] = pltpu.async_copy(
                    rbufs[b], xg_hbm.at[pl.ds(le * k + i * ch, ch)], osems[b])
            out_cps[0].wait()
            out_cps[1].wait()
        pltpu.sync_copy(hist_v, fpart_hbm.at[pl.ds(wid * n, n)])

    return place_k


def _unpack_bf16(u32):
    h1 = lax.bitcast_convert_type((u32 >> 16).astype(jnp.uint16), jnp.bfloat16)
    h2 = lax.bitcast_convert_type(
        (u32 & jnp.uint32(0xFFFF)).astype(jnp.uint16), jnp.bfloat16)
    return jnp.concatenate([h1, h2], axis=1)


def _experts_half_body(xg_ref, w1_ref, w2_ref, hin_ref, out_ref):
    del hin_ref
    x = _unpack_bf16(lax.bitcast_convert_type(xg_ref[...], jnp.uint32))
    h = lax.dot_general(x, w1_ref[0].astype(jnp.bfloat16),
                        (((1,), (1,)), ((), ())),
                        preferred_element_type=jnp.float32)
    h = jax.nn.gelu(h)
    out_ref[...] = lax.dot_general(h, w2_ref[0], (((1,), (1,)), ((), ())),
                                   preferred_element_type=jnp.float32)


def _experts_half(xg_half, expert_W1, expert_W2, h_prev, e_lo, nexp, k):
    e, d, c = expert_W1.shape
    rb = 256
    nrb = k // rb
    return pl.pallas_call(
        _experts_half_body,
        grid=(nexp, nrb),
        in_specs=[
            pl.BlockSpec((rb, c // 2), lambda i, j: (i * nrb + j, 0)),
            pl.BlockSpec((1, d, c), lambda i, j: (e_lo + i, 0, 0)),
            pl.BlockSpec((1, c, d), lambda i, j: (e_lo + i, 0, 0)),
            pl.BlockSpec(memory_space=pltpu.HBM),
        ],
        out_specs=pl.BlockSpec(
            (rb, c), lambda i, j: ((e_lo + i) * nrb + j, 0)),
        out_shape=jax.ShapeDtypeStruct((e * k, c), jnp.float32),
        input_output_aliases={3: 0},
        compiler_params=pltpu.CompilerParams(
            dimension_semantics=("arbitrary", "arbitrary")),
    )(xg_half, expert_W1, expert_W2, h_prev)


def _fanout_body(fp0_ref, fp1_ref, fo_ref):
    fo_ref[...] = (jnp.sum(fp0_ref[...], axis=0, keepdims=True)
                   + jnp.sum(fp1_ref[...], axis=0, keepdims=True))


def _fanout(fp0, fp1):
    nw, n = fp0.shape
    return pl.pallas_call(
        _fanout_body,
        in_specs=[pl.BlockSpec((nw, n), lambda: (0, 0)),
                  pl.BlockSpec((nw, n), lambda: (0, 0))],
        out_specs=pl.BlockSpec((1, n), lambda: (0, 0)),
        out_shape=jax.ShapeDtypeStruct((1, n), jnp.float32),
    )(fp0, fp1)


def kernel(x, router_W, expert_W1, expert_W2):
    B, T, C = x.shape
    n_tokens = B * T
    k = n_tokens // EXPANSION
    x_flat = x.reshape(-1, C)

    allwt, keys, xbf = _router(x_flat, router_W)
    t_bcast = _threshold(keys, k)
    t_sc = t_bcast[:, :16].reshape(-1)
    keysc, idxc = _make_sc_compact(n_tokens)(keys.reshape(-1), t_sc)
    ranks = _ranks(keysc.reshape(N_EXPERTS, CAP))

    zi = jnp.zeros((n_tokens,), jnp.int32)
    zf = jnp.zeros((n_tokens,), jnp.float32)
    half = N_EXPERTS // 2
    ranks_f = ranks.reshape(-1)
    allwt_f = allwt.reshape(-1)
    li0, w0, fp0, xg0 = _make_sc_place_gather(
        n_tokens, k, C // 2, 0, half)(idxc, ranks_f, allwt_f, zi, zf, xbf)
    li1, w1h, fp1, xg1 = _make_sc_place_gather(
        n_tokens, k, C // 2, half, half)(idxc, ranks_f, allwt_f, zi, zf, xbf)
    h_seed = jnp.zeros((N_EXPERTS * k, C), jnp.float32)
    h_a = _experts_half(xg0, expert_W1, expert_W2, h_seed, 0, half, k)
    h_flat = _experts_half(xg1, expert_W1, expert_W2, h_a, half, half, k)
    fo = _fanout(fp0.reshape(_NW, n_tokens), fp1.reshape(_NW, n_tokens))
    local_indices = jnp.concatenate([li0, li1])
    weights_flat = jnp.concatenate([w0, w1h])
    return h_flat, local_indices, weights_flat, fo.reshape(n_tokens)


# speedup vs baseline: 1.1374x; 1.1374x over previous
import functools

import numpy as np
import jax
import jax.numpy as jnp
from jax import lax
from jax.experimental import pallas as pl
from jax.experimental.pallas import tpu as pltpu
from jax.experimental.pallas import tpu_sc as plsc

N_EXPERTS = 64
EXPANSION = 8
CAP = 1152

try:
    _SC_INFO = plsc.get_sparse_core_info()
    _NC, _NS = _SC_INFO.num_cores, _SC_INFO.num_subcores
except Exception:
    _NC, _NS = 2, 16
_NW = _NC * _NS
_EPW = N_EXPERTS // _NW

_INT_MIN = np.int32(-2147483648)


def _router_body(x_ref, rw_ref, allwt_ref, keys_ref, xb_ref):
    lgt = lax.dot_general(rw_ref[...].astype(jnp.bfloat16),
                          x_ref[...].astype(jnp.bfloat16),
                          (((1,), (1,)), ((), ())),
                          preferred_element_type=jnp.float32)
    m = jnp.max(lgt, axis=0, keepdims=True)
    e = jnp.exp(lgt - m)
    allwt_ref[...] = e / jnp.sum(e, axis=0, keepdims=True)
    u = lax.bitcast_convert_type(lgt, jnp.uint32)
    neg = u >= jnp.uint32(0x80000000)
    up = jnp.where(neg, ~u, u | jnp.uint32(0x80000000))
    keys_ref[...] = lax.bitcast_convert_type(up ^ jnp.uint32(0x80000000),
                                             jnp.int32)
    xb = x_ref[...].astype(jnp.bfloat16)
    c = xb.shape[1]
    a = lax.bitcast_convert_type(xb[:, : c // 2], jnp.uint16).astype(jnp.uint32)
    b = lax.bitcast_convert_type(xb[:, c // 2 :], jnp.uint16).astype(jnp.uint32)
    xb_ref[...] = lax.bitcast_convert_type((a << 16) | b, jnp.int32)


def _router(x_flat, router_W):
    n, c = x_flat.shape
    blk = 1024
    return pl.pallas_call(
        _router_body,
        grid=(n // blk,),
        in_specs=[
            pl.BlockSpec((blk, c), lambda i: (i, 0)),
            pl.BlockSpec((N_EXPERTS, c), lambda i: (0, 0)),
        ],
        out_specs=[
            pl.BlockSpec((N_EXPERTS, blk), lambda i: (0, i)),
            pl.BlockSpec((N_EXPERTS, blk), lambda i: (0, i)),
            pl.BlockSpec((blk, c // 2), lambda i: (i, 0)),
        ],
        out_shape=[
            jax.ShapeDtypeStruct((N_EXPERTS, n), jnp.float32),
            jax.ShapeDtypeStruct((N_EXPERTS, n), jnp.int32),
            jax.ShapeDtypeStruct((n, c // 2), jnp.int32),
        ],
    )(x_flat, router_W)


def _thresh_body(k, keys_ref, t_ref):
    keys = keys_ref[...]
    t = jnp.full((N_EXPERTS, 1), _INT_MIN, jnp.int32)
    for b in range(31, -1, -1):
        cand = t + np.array(1 << b, dtype=np.uint32).view(np.int32)
        cnt = jnp.sum((keys >= cand).astype(jnp.int32), axis=1, keepdims=True)
        t = jnp.where(cnt >= k, cand, t)
    t_ref[...] = jnp.broadcast_to(t, (N_EXPERTS, 128))


def _threshold(keys, k):
    n = keys.shape[1]
    return pl.pallas_call(
        functools.partial(_thresh_body, k),
        in_specs=[pl.BlockSpec((N_EXPERTS, n), lambda: (0, 0))],
        out_specs=pl.BlockSpec((N_EXPERTS, 128), lambda: (0, 0)),
        out_shape=jax.ShapeDtypeStruct((N_EXPERTS, 128), jnp.int32),
    )(keys)


def _make_sc_compact(n):
    n_vregs = n // 16
    mesh = plsc.VectorSubcoreMesh(core_axis_name="c", subcore_axis_name="s")

    @functools.partial(
        pl.kernel,
        mesh=mesh,
        compiler_params=pltpu.CompilerParams(needs_layout_passes=False),
        out_type=(
            jax.ShapeDtypeStruct((N_EXPERTS * CAP,), jnp.int32),
            jax.ShapeDtypeStruct((N_EXPERTS * CAP,), jnp.int32),
        ),
        scratch_types=[
            pltpu.VMEM((n,), jnp.int32),
            pltpu.VMEM((16,), jnp.int32),
            pltpu.VMEM((CAP,), jnp.int32),
            pltpu.VMEM((CAP,), jnp.int32),
        ],
    )
    def compact_k(keys_hbm, t_hbm, keysc_hbm, idxc_hbm, keys_v, t_v, kc_v, ic_v):
        wid = lax.axis_index("s") * _NC + lax.axis_index("c")
        for t in range(_EPW):
            e = wid * _EPW + t
            pltpu.sync_copy(keys_hbm.at[pl.ds(e * n, n)], keys_v)
            pltpu.sync_copy(t_hbm.at[pl.ds(e * 16, 16)], t_v)
            tvec = t_v[...]

            def fill(j, carry):
                kc_v[pl.ds(j * 16, 16)] = jnp.full((16,), _INT_MIN, jnp.int32)
                ic_v[pl.ds(j * 16, 16)] = jnp.full((16,), 0x7FFFFFFF, jnp.int32)
                return carry

            lax.fori_loop(0, CAP // 16, fill, 0)

            def step(i, cnt):
                kv = keys_v[pl.ds(i * 16, 16)]
                m = (kv >= tvec) & (cnt < CAP - 16)
                iv = lax.iota(jnp.int32, 16) + i * 16
                cs = plsc.cumsum(m.astype(jnp.int32))
                dest = cnt + cs - 1
                plsc.store_scatter(kc_v, [dest], kv, mask=m)
                plsc.store_scatter(ic_v, [dest], iv, mask=m)
                return cnt + jnp.max(cs)

            lax.fori_loop(0, n_vregs, step, jnp.int32(0))
            pltpu.sync_copy(kc_v, keysc_hbm.at[pl.ds(e * CAP, CAP)])
            pltpu.sync_copy(ic_v, idxc_hbm.at[pl.ds(e * CAP, CAP)])

    return compact_k


def _rank_body(keys_ref, rank_ref):
    keys = keys_ref[0, 0]
    kc = keys.reshape(CAP, 1)
    kr = keys.reshape(1, CAP)
    row = lax.broadcasted_iota(jnp.int32, (CAP, CAP), 0)
    col = lax.broadcasted_iota(jnp.int32, (CAP, CAP), 1)
    a = (kr > kc) | ((kr == kc) & (col < row))
    rank_ref[0, 0] = jnp.sum(a.astype(jnp.int32), axis=1)


def _ranks(keysc):
    return pl.pallas_call(
        _rank_body,
        grid=(N_EXPERTS,),
        in_specs=[pl.BlockSpec((1, 1, CAP), lambda i: (i, 0, 0))],
        out_specs=pl.BlockSpec((1, 1, CAP), lambda i: (i, 0, 0)),
        out_shape=jax.ShapeDtypeStruct((N_EXPERTS, 1, CAP), jnp.int32),
    )(keysc.reshape(N_EXPERTS, 1, CAP))


def _make_sc_place_gather(n, k, cw, e_lo, nexp):
    ch = 64
    nch = k // ch
    mesh = plsc.VectorSubcoreMesh(core_axis_name="c", subcore_axis_name="s")

    @functools.partial(
        pl.kernel,
        mesh=mesh,
        compiler_params=pltpu.CompilerParams(needs_layout_passes=False),
        out_type=(
            jax.ShapeDtypeStruct((nexp * k,), jnp.int32),
            jax.ShapeDtypeStruct((nexp * k,), jnp.float32),
            jax.ShapeDtypeStruct((_NW * n,), jnp.float32),
            jax.ShapeDtypeStruct((nexp * k, cw), jnp.int32),
        ),
        scratch_types=[
            pltpu.VMEM((CAP,), jnp.int32),
            pltpu.VMEM((CAP,), jnp.int32),
            pltpu.VMEM((k,), jnp.int32),
            pltpu.VMEM((n,), jnp.float32),
            pltpu.VMEM((k,), jnp.float32),
            pltpu.VMEM((n,), jnp.float32),
            pltpu.VMEM((64, cw), jnp.int32),
            pltpu.VMEM((64, cw), jnp.int32),
            pltpu.SemaphoreType.DMA,
            pltpu.SemaphoreType.DMA,
            pltpu.SemaphoreType.DMA,
        ],
    )
    def place_k(idxc_hbm, rank_hbm, allwt_hbm, zi_hbm, zf_hbm, xb_hbm,
                lidx_hbm, wflat_hbm, fpart_hbm, xg_hbm,
                ic_v, rk_v, tk_v, aw_v, w_v, hist_v, rb0, rb1, gsem, os0, os1):
        wid = lax.axis_index("s") * _NC + lax.axis_index("c")
        pltpu.sync_copy(zf_hbm, hist_v)
        ones = jnp.full((16,), 1.0, jnp.float32)
        rbufs = (rb0, rb1)
        osems = (os0, os1)
        epw = nexp // _NW
        for t in range(epw):
            le = wid * epw + t
            e = e_lo + le
            pltpu.sync_copy(idxc_hbm.at[pl.ds(e * CAP, CAP)], ic_v)
            pltpu.sync_copy(rank_hbm.at[pl.ds(e * CAP, CAP)], rk_v)
            pltpu.sync_copy(zi_hbm.at[pl.ds(0, k)], tk_v)

            def place(j, carry):
                r = rk_v[pl.ds(j * 16, 16)]
                iv = ic_v[pl.ds(j * 16, 16)]
                m = r < k
                plsc.store_scatter(tk_v, [r], iv, mask=m)
                return carry

            lax.fori_loop(0, CAP // 16, place, 0)
            pltpu.sync_copy(tk_v, lidx_hbm.at[pl.ds(le * k, k)])
            pltpu.sync_copy(allwt_hbm.at[pl.ds(e * n, n)], aw_v)

            def wgather(j, carry):
                tok = tk_v[pl.ds(j * 16, 16)]
                w_v[pl.ds(j * 16, 16)] = plsc.load_gather(aw_v, [tok])
                plsc.addupdate_scatter(hist_v, [tok], ones)
                return carry

            lax.fori_loop(0, k // 16, wgather, 0)
            pltpu.sync_copy(w_v, wflat_hbm.at[pl.ds(le * k, k)])

            ch = 64
            out_cps = [None, None]
            for i in range(k // ch):
                b = i & 1
                if out_cps[b] is not None:
                    out_cps[b].wait()
                pltpu.async_copy(xb_hbm.at[tk_v.at[pl.ds(i * ch, ch)]],
                                 rbufs[b], gsem).wait()
                out_cps[b] = pltpu.async_copy(
                    rbufs[b], xg_hbm.at[pl.ds(le * k + i * ch, ch)], osems[b])
            out_cps[0].wait()
            out_cps[1].wait()
        pltpu.sync_copy(hist_v, fpart_hbm.at[pl.ds(wid * n, n)])

    return place_k


def _unpack_bf16(u32):
    h1 = lax.bitcast_convert_type((u32 >> 16).astype(jnp.uint16), jnp.bfloat16)
    h2 = lax.bitcast_convert_type(
        (u32 & jnp.uint32(0xFFFF)).astype(jnp.uint16), jnp.bfloat16)
    return jnp.concatenate([h1, h2], axis=1)


def _experts_half_body(xg_ref, w1_ref, w2_ref, *refs):
    out_ref = refs[-1]
    x = _unpack_bf16(lax.bitcast_convert_type(xg_ref[...], jnp.uint32))
    h = lax.dot_general(x, w1_ref[0].astype(jnp.bfloat16),
                        (((1,), (1,)), ((), ())),
                        preferred_element_type=jnp.float32)
    h = jax.nn.gelu(h)
    out_ref[...] = lax.dot_general(h, w2_ref[0], (((1,), (1,)), ((), ())),
                                   preferred_element_type=jnp.float32)


def _experts_half(xg_half, expert_W1, expert_W2, h_prev, e_lo, nexp, k):
    e, d, c = expert_W1.shape
    rb = 256
    nrb = k // rb
    in_specs = [
        pl.BlockSpec((rb, c // 2), lambda i, j: (i * nrb + j, 0)),
        pl.BlockSpec((1, d, c), lambda i, j: (e_lo + i, 0, 0)),
        pl.BlockSpec((1, c, d), lambda i, j: (e_lo + i, 0, 0)),
    ]
    args = [xg_half, expert_W1, expert_W2]
    aliases = {}
    if h_prev is not None:
        in_specs.append(pl.BlockSpec(memory_space=pltpu.HBM))
        args.append(h_prev)
        aliases = {3: 0}
    return pl.pallas_call(
        _experts_half_body,
        grid=(nexp, nrb),
        in_specs=in_specs,
        out_specs=pl.BlockSpec(
            (rb, c), lambda i, j: ((e_lo + i) * nrb + j, 0)),
        out_shape=jax.ShapeDtypeStruct((e * k, c), jnp.float32),
        input_output_aliases=aliases,
        compiler_params=pltpu.CompilerParams(
            dimension_semantics=("arbitrary", "arbitrary")),
    )(*args)


def _fanout_body(fp0_ref, fp1_ref, fo_ref):
    fo_ref[...] = (jnp.sum(fp0_ref[...], axis=0, keepdims=True)
                   + jnp.sum(fp1_ref[...], axis=0, keepdims=True))


def _fanout(fp0, fp1):
    nw, n = fp0.shape
    return pl.pallas_call(
        _fanout_body,
        in_specs=[pl.BlockSpec((nw, n), lambda: (0, 0)),
                  pl.BlockSpec((nw, n), lambda: (0, 0))],
        out_specs=pl.BlockSpec((1, n), lambda: (0, 0)),
        out_shape=jax.ShapeDtypeStruct((1, n), jnp.float32),
    )(fp0, fp1)


def kernel(x, router_W, expert_W1, expert_W2):
    B, T, C = x.shape
    n_tokens = B * T
    k = n_tokens // EXPANSION
    x_flat = x.reshape(-1, C)

    allwt, keys, xbf = _router(x_flat, router_W)
    t_bcast = _threshold(keys, k)
    t_sc = t_bcast[:, :16].reshape(-1)
    keysc, idxc = _make_sc_compact(n_tokens)(keys.reshape(-1), t_sc)
    ranks = _ranks(keysc.reshape(N_EXPERTS, CAP))

    zi = jnp.zeros((n_tokens,), jnp.int32)
    zf = jnp.zeros((n_tokens,), jnp.float32)
    half = N_EXPERTS // 2
    ranks_f = ranks.reshape(-1)
    allwt_f = allwt.reshape(-1)
    li0, w0, fp0, xg0 = _make_sc_place_gather(
        n_tokens, k, C // 2, 0, half)(idxc, ranks_f, allwt_f, zi, zf, xbf)
    li1, w1h, fp1, xg1 = _make_sc_place_gather(
        n_tokens, k, C // 2, half, half)(idxc, ranks_f, allwt_f, zi, zf, xbf)
    h_a = _experts_half(xg0, expert_W1, expert_W2, None, 0, half, k)
    h_flat = _experts_half(xg1, expert_W1, expert_W2, h_a, half, half, k)
    fo = _fanout(fp0.reshape(_NW, n_tokens), fp1.reshape(_NW, n_tokens))
    local_indices = jnp.concatenate([li0, li1])
    weights_flat = jnp.concatenate([w0, w1h])
    return h_flat, local_indices, weights_flat, fo.reshape(n_tokens)


# speedup vs baseline: 1.1745x; 1.0326x over previous
import functools

import numpy as np
import jax
import jax.numpy as jnp
from jax import lax
from jax.experimental import pallas as pl
from jax.experimental.pallas import tpu as pltpu
from jax.experimental.pallas import tpu_sc as plsc

N_EXPERTS = 64
EXPANSION = 8
CAP = 1152

try:
    _SC_INFO = plsc.get_sparse_core_info()
    _NC, _NS = _SC_INFO.num_cores, _SC_INFO.num_subcores
except Exception:
    _NC, _NS = 2, 16
_NW = _NC * _NS
_EPW = N_EXPERTS // _NW

_INT_MIN = np.int32(-2147483648)


def _router_body(nsteps, k, x_ref, rw_ref, allwt_ref, keys_ref, xb_ref,
                 t_ref, keys_acc):
    lgt = lax.dot_general(rw_ref[...].astype(jnp.bfloat16),
                          x_ref[...].astype(jnp.bfloat16),
                          (((1,), (1,)), ((), ())),
                          preferred_element_type=jnp.float32)
    m = jnp.max(lgt, axis=0, keepdims=True)
    e = jnp.exp(lgt - m)
    allwt_ref[...] = e / jnp.sum(e, axis=0, keepdims=True)
    u = lax.bitcast_convert_type(lgt, jnp.uint32)
    neg = u >= jnp.uint32(0x80000000)
    up = jnp.where(neg, ~u, u | jnp.uint32(0x80000000))
    ikeys = lax.bitcast_convert_type(up ^ jnp.uint32(0x80000000), jnp.int32)
    keys_ref[...] = ikeys
    blk = ikeys.shape[1]
    keys_acc[:, pl.ds(pl.program_id(0) * blk, blk)] = ikeys

    @pl.when(pl.program_id(0) == nsteps - 1)
    def _():
        keys = keys_acc[...]
        t = jnp.full((N_EXPERTS, 1), _INT_MIN, jnp.int32)
        for b in range(31, -1, -1):
            cand = t + np.array(1 << b, dtype=np.uint32).view(np.int32)
            cnt = jnp.sum((keys >= cand).astype(jnp.int32), axis=1,
                          keepdims=True)
            t = jnp.where(cnt >= k, cand, t)
        t_ref[...] = jnp.broadcast_to(t, (N_EXPERTS, 128))

    xb = x_ref[...].astype(jnp.bfloat16)
    c = xb.shape[1]
    a = lax.bitcast_convert_type(xb[:, : c // 2], jnp.uint16).astype(jnp.uint32)
    b = lax.bitcast_convert_type(xb[:, c // 2 :], jnp.uint16).astype(jnp.uint32)
    xb_ref[...] = lax.bitcast_convert_type((a << 16) | b, jnp.int32)


def _router(x_flat, router_W, k):
    n, c = x_flat.shape
    blk = 1024
    return pl.pallas_call(
        functools.partial(_router_body, n // blk, k),
        grid=(n // blk,),
        in_specs=[
            pl.BlockSpec((blk, c), lambda i: (i, 0)),
            pl.BlockSpec((N_EXPERTS, c), lambda i: (0, 0)),
        ],
        out_specs=[
            pl.BlockSpec((N_EXPERTS, blk), lambda i: (0, i)),
            pl.BlockSpec((N_EXPERTS, blk), lambda i: (0, i)),
            pl.BlockSpec((blk, c // 2), lambda i: (i, 0)),
            pl.BlockSpec((N_EXPERTS, 128), lambda i: (0, 0)),
        ],
        out_shape=[
            jax.ShapeDtypeStruct((N_EXPERTS, n), jnp.float32),
            jax.ShapeDtypeStruct((N_EXPERTS, n), jnp.int32),
            jax.ShapeDtypeStruct((n, c // 2), jnp.int32),
            jax.ShapeDtypeStruct((N_EXPERTS, 128), jnp.int32),
        ],
        scratch_shapes=[pltpu.VMEM((N_EXPERTS, n), jnp.int32)],
    )(x_flat, router_W)


def _thresh_body(k, keys_ref, t_ref):
    keys = keys_ref[...]
    t = jnp.full((N_EXPERTS, 1), _INT_MIN, jnp.int32)
    for b in range(31, -1, -1):
        cand = t + np.array(1 << b, dtype=np.uint32).view(np.int32)
        cnt = jnp.sum((keys >= cand).astype(jnp.int32), axis=1, keepdims=True)
        t = jnp.where(cnt >= k, cand, t)
    t_ref[...] = jnp.broadcast_to(t, (N_EXPERTS, 128))


def _threshold(keys, k):
    n = keys.shape[1]
    return pl.pallas_call(
        functools.partial(_thresh_body, k),
        in_specs=[pl.BlockSpec((N_EXPERTS, n), lambda: (0, 0))],
        out_specs=pl.BlockSpec((N_EXPERTS, 128), lambda: (0, 0)),
        out_shape=jax.ShapeDtypeStruct((N_EXPERTS, 128), jnp.int32),
    )(keys)


def _make_sc_compact(n):
    n_vregs = n // 16
    mesh = plsc.VectorSubcoreMesh(core_axis_name="c", subcore_axis_name="s")

    @functools.partial(
        pl.kernel,
        mesh=mesh,
        compiler_params=pltpu.CompilerParams(needs_layout_passes=False),
        out_type=(
            jax.ShapeDtypeStruct((N_EXPERTS * CAP,), jnp.int32),
            jax.ShapeDtypeStruct((N_EXPERTS * CAP,), jnp.int32),
        ),
        scratch_types=[
            pltpu.VMEM((n,), jnp.int32),
            pltpu.VMEM((16,), jnp.int32),
            pltpu.VMEM((CAP,), jnp.int32),
            pltpu.VMEM((CAP,), jnp.int32),
        ],
    )
    def compact_k(keys_hbm, t_hbm, keysc_hbm, idxc_hbm, keys_v, t_v, kc_v, ic_v):
        wid = lax.axis_index("s") * _NC + lax.axis_index("c")
        for t in range(_EPW):
            e = wid * _EPW + t
            pltpu.sync_copy(keys_hbm.at[pl.ds(e * n, n)], keys_v)
            pltpu.sync_copy(t_hbm.at[pl.ds(e * 16, 16)], t_v)
            tvec = t_v[...]

            def fill(j, carry):
                kc_v[pl.ds(j * 16, 16)] = jnp.full((16,), _INT_MIN, jnp.int32)
                ic_v[pl.ds(j * 16, 16)] = jnp.full((16,), 0x7FFFFFFF, jnp.int32)
                return carry

            lax.fori_loop(0, CAP // 16, fill, 0)

            def step(i, cnt):
                kv = keys_v[pl.ds(i * 16, 16)]
                m = (kv >= tvec) & (cnt < CAP - 16)
                iv = lax.iota(jnp.int32, 16) + i * 16
                cs = plsc.cumsum(m.astype(jnp.int32))
                dest = cnt + cs - 1
                plsc.store_scatter(kc_v, [dest], kv, mask=m)
                plsc.store_scatter(ic_v, [dest], iv, mask=m)
                return cnt + jnp.max(cs)

            lax.fori_loop(0, n_vregs, step, jnp.int32(0))
            pltpu.sync_copy(kc_v, keysc_hbm.at[pl.ds(e * CAP, CAP)])
            pltpu.sync_copy(ic_v, idxc_hbm.at[pl.ds(e * CAP, CAP)])

    return compact_k


def _rank_body(keys_ref, rank_ref):
    keys = keys_ref[0, 0]
    kc = keys.reshape(CAP, 1)
    kr = keys.reshape(1, CAP)
    row = lax.broadcasted_iota(jnp.int32, (CAP, CAP), 0)
    col = lax.broadcasted_iota(jnp.int32, (CAP, CAP), 1)
    a = (kr > kc) | ((kr == kc) & (col < row))
    rank_ref[0, 0] = jnp.sum(a.astype(jnp.int32), axis=1)


def _ranks(keysc):
    return pl.pallas_call(
        _rank_body,
        grid=(N_EXPERTS,),
        in_specs=[pl.BlockSpec((1, 1, CAP), lambda i: (i, 0, 0))],
        out_specs=pl.BlockSpec((1, 1, CAP), lambda i: (i, 0, 0)),
        out_shape=jax.ShapeDtypeStruct((N_EXPERTS, 1, CAP), jnp.int32),
    )(keysc.reshape(N_EXPERTS, 1, CAP))


def _make_sc_place_gather(n, k, cw):
    ch = 64
    nch = k // ch
    mesh = plsc.VectorSubcoreMesh(core_axis_name="c", subcore_axis_name="s")

    @functools.partial(
        pl.kernel,
        mesh=mesh,
        compiler_params=pltpu.CompilerParams(needs_layout_passes=False),
        out_type=(
            jax.ShapeDtypeStruct((N_EXPERTS * k,), jnp.int32),
            jax.ShapeDtypeStruct((N_EXPERTS * k,), jnp.float32),
            jax.ShapeDtypeStruct((_NW * n,), jnp.float32),
            jax.ShapeDtypeStruct((N_EXPERTS * k, cw), jnp.int32),
        ),
        scratch_types=[
            pltpu.VMEM((CAP,), jnp.int32),
            pltpu.VMEM((CAP,), jnp.int32),
            pltpu.VMEM((k,), jnp.int32),
            pltpu.VMEM((n,), jnp.float32),
            pltpu.VMEM((k,), jnp.float32),
            pltpu.VMEM((n,), jnp.float32),
            pltpu.VMEM((64, cw), jnp.int32),
            pltpu.VMEM((64, cw), jnp.int32),
            pltpu.SemaphoreType.DMA,
            pltpu.SemaphoreType.DMA,
            pltpu.SemaphoreType.DMA,
        ],
    )
    def place_k(idxc_hbm, rank_hbm, allwt_hbm, zi_hbm, zf_hbm, xb_hbm,
                lidx_hbm, wflat_hbm, fpart_hbm, xg_hbm,
                ic_v, rk_v, tk_v, aw_v, w_v, hist_v, rb0, rb1, gsem, os0, os1):
        wid = lax.axis_index("s") * _NC + lax.axis_index("c")
        pltpu.sync_copy(zf_hbm, hist_v)
        ones = jnp.full((16,), 1.0, jnp.float32)
        rbufs = (rb0, rb1)
        osems = (os0, os1)
        for t in range(_EPW):
            e = wid * _EPW + t
            pltpu.sync_copy(idxc_hbm.at[pl.ds(e * CAP, CAP)], ic_v)
            pltpu.sync_copy(rank_hbm.at[pl.ds(e * CAP, CAP)], rk_v)
            pltpu.sync_copy(zi_hbm.at[pl.ds(0, k)], tk_v)

            def place(j, carry):
                r = rk_v[pl.ds(j * 16, 16)]
                iv = ic_v[pl.ds(j * 16, 16)]
                m = r < k
                plsc.store_scatter(tk_v, [r], iv, mask=m)
                return carry

            lax.fori_loop(0, CAP // 16, place, 0)
            pltpu.sync_copy(tk_v, lidx_hbm.at[pl.ds(e * k, k)])
            pltpu.sync_copy(allwt_hbm.at[pl.ds(e * n, n)], aw_v)

            def wgather(j, carry):
                tok = tk_v[pl.ds(j * 16, 16)]
                w_v[pl.ds(j * 16, 16)] = plsc.load_gather(aw_v, [tok])
                plsc.addupdate_scatter(hist_v, [tok], ones)
                return carry

            lax.fori_loop(0, k // 16, wgather, 0)
            pltpu.sync_copy(w_v, wflat_hbm.at[pl.ds(e * k, k)])

            ch = 64
            out_cps = [None, None]
            for i in range(k // ch):
                b = i & 1
                if out_cps[b] is not None:
                    out_cps[b].wait()
                pltpu.async_copy(xb_hbm.at[tk_v.at[pl.ds(i * ch, ch)]],
                                 rbufs[b], gsem).wait()
                out_cps[b] = pltpu.async_copy(
                    rbufs[b], xg_hbm.at[pl.ds(e * k + i * ch, ch)], osems[b])
            out_cps[0].wait()
            out_cps[1].wait()
        pltpu.sync_copy(hist_v, fpart_hbm.at[pl.ds(wid * n, n)])

    return place_k


def _experts_body(xg_ref, w1_ref, w2_ref, fp_ref, out_ref, fo_ref):
    u = lax.bitcast_convert_type(xg_ref[...], jnp.uint32)
    h1 = lax.bitcast_convert_type((u >> 16).astype(jnp.uint16), jnp.bfloat16)
    h2 = lax.bitcast_convert_type(
        (u & jnp.uint32(0xFFFF)).astype(jnp.uint16), jnp.bfloat16)
    x = jnp.concatenate([h1, h2], axis=1)
    h = lax.dot_general(x, w1_ref[0].astype(jnp.bfloat16),
                        (((1,), (1,)), ((), ())),
                        preferred_element_type=jnp.float32)
    h = jax.nn.gelu(h)
    out_ref[...] = lax.dot_general(h, w2_ref[0], (((1,), (1,)), ((), ())),
                                   preferred_element_type=jnp.float32)
    @pl.when((pl.program_id(0) == 0) & (pl.program_id(1) == 0))
    def _():
        fo_ref[...] = jnp.sum(fp_ref[...], axis=0, keepdims=True)


def _experts(xg, expert_W1, expert_W2, fpart, k):
    e, d, c = expert_W1.shape
    n = fpart.shape[1]
    rb = 256
    nrb = k // rb
    return pl.pallas_call(
        _experts_body,
        grid=(e, nrb),
        in_specs=[
            pl.BlockSpec((rb, c // 2), lambda i, j: (i * nrb + j, 0)),
            pl.BlockSpec((1, d, c), lambda i, j: (i, 0, 0)),
            pl.BlockSpec((1, c, d), lambda i, j: (i, 0, 0)),
            pl.BlockSpec((_NW, n), lambda i, j: (0, 0)),
        ],
        out_specs=[
            pl.BlockSpec((rb, c), lambda i, j: (i * nrb + j, 0)),
            pl.BlockSpec((1, n), lambda i, j: (0, 0)),
        ],
        out_shape=[
            jax.ShapeDtypeStruct((e * k, c), jnp.float32),
            jax.ShapeDtypeStruct((1, n), jnp.float32),
        ],
        compiler_params=pltpu.CompilerParams(
            dimension_semantics=("arbitrary", "arbitrary")),
    )(xg, expert_W1, expert_W2, fpart)


def kernel(x, router_W, expert_W1, expert_W2):
    B, T, C = x.shape
    n_tokens = B * T
    k = n_tokens // EXPANSION
    x_flat = x.reshape(-1, C)

    allwt, keys, xbf, t_bcast = _router(x_flat, router_W, k)
    t_sc = t_bcast[:, :16].reshape(-1)
    keysc, idxc = _make_sc_compact(n_tokens)(keys.reshape(-1), t_sc)
    ranks = _ranks(keysc.reshape(N_EXPERTS, CAP))

    zi = jnp.zeros((n_tokens,), jnp.int32)
    zf = jnp.zeros((n_tokens,), jnp.float32)
    local_indices, weights_flat, fpart, xg = _make_sc_place_gather(
        n_tokens, k, C // 2)(idxc, ranks.reshape(-1), allwt.reshape(-1),
                             zi, zf, xbf)
    h_flat, fo = _experts(xg, expert_W1, expert_W2,
                          fpart.reshape(_NW, n_tokens), k)
    return h_flat, local_indices, weights_flat, fo.reshape(n_tokens)


# speedup vs baseline: 1.3577x; 1.1560x over previous
import functools

import numpy as np
import jax
import jax.numpy as jnp
from jax import lax
from jax.experimental import pallas as pl
from jax.experimental.pallas import tpu as pltpu
from jax.experimental.pallas import tpu_sc as plsc

N_EXPERTS = 64
EXPANSION = 8
CAP = 1152

try:
    _SC_INFO = plsc.get_sparse_core_info()
    _NC, _NS = _SC_INFO.num_cores, _SC_INFO.num_subcores
except Exception:
    _NC, _NS = 2, 16
_NW = _NC * _NS
_EPW = N_EXPERTS // _NW

_INT_MIN = np.int32(-2147483648)


def _router_body(nsteps, k, x_ref, rw_ref, allwt_ref, keys_ref, xb_ref,
                 t_ref, keys_acc):
    lgt = lax.dot_general(rw_ref[...].astype(jnp.bfloat16),
                          x_ref[...].astype(jnp.bfloat16),
                          (((1,), (1,)), ((), ())),
                          preferred_element_type=jnp.float32)
    m = jnp.max(lgt, axis=0, keepdims=True)
    e = jnp.exp(lgt - m)
    allwt_ref[...] = e / jnp.sum(e, axis=0, keepdims=True)
    u = lax.bitcast_convert_type(lgt, jnp.uint32)
    neg = u >= jnp.uint32(0x80000000)
    up = jnp.where(neg, ~u, u | jnp.uint32(0x80000000))
    ikeys = lax.bitcast_convert_type(up ^ jnp.uint32(0x80000000), jnp.int32)
    keys_ref[...] = ikeys
    blk = ikeys.shape[1]
    keys_acc[:, pl.ds(pl.program_id(0) * blk, blk)] = ikeys

    @pl.when(pl.program_id(0) == nsteps - 1)
    def _():
        keys = keys_acc[...]
        t = jnp.full((N_EXPERTS, 1), _INT_MIN, jnp.int32)
        for b in range(31, -1, -1):
            cand = t + np.array(1 << b, dtype=np.uint32).view(np.int32)
            cnt = jnp.sum((keys >= cand).astype(jnp.int32), axis=1,
                          keepdims=True)
            t = jnp.where(cnt >= k, cand, t)
        t_ref[...] = jnp.broadcast_to(t, (N_EXPERTS, 128))

    xb = x_ref[...].astype(jnp.bfloat16)
    c = xb.shape[1]
    a = lax.bitcast_convert_type(xb[:, : c // 2], jnp.uint16).astype(jnp.uint32)
    b = lax.bitcast_convert_type(xb[:, c // 2 :], jnp.uint16).astype(jnp.uint32)
    xb_ref[...] = lax.bitcast_convert_type((a << 16) | b, jnp.int32)


def _router(x_flat, router_W, k):
    n, c = x_flat.shape
    blk = 1024
    return pl.pallas_call(
        functools.partial(_router_body, n // blk, k),
        grid=(n // blk,),
        in_specs=[
            pl.BlockSpec((blk, c), lambda i: (i, 0)),
            pl.BlockSpec((N_EXPERTS, c), lambda i: (0, 0)),
        ],
        out_specs=[
            pl.BlockSpec((N_EXPERTS, blk), lambda i: (0, i)),
            pl.BlockSpec((N_EXPERTS, blk), lambda i: (0, i)),
            pl.BlockSpec((blk, c // 2), lambda i: (i, 0)),
            pl.BlockSpec((N_EXPERTS, 128), lambda i: (0, 0)),
        ],
        out_shape=[
            jax.ShapeDtypeStruct((N_EXPERTS, n), jnp.float32),
            jax.ShapeDtypeStruct((N_EXPERTS, n), jnp.int32),
            jax.ShapeDtypeStruct((n, c // 2), jnp.int32),
            jax.ShapeDtypeStruct((N_EXPERTS, 128), jnp.int32),
        ],
        scratch_shapes=[pltpu.VMEM((N_EXPERTS, n), jnp.int32)],
    )(x_flat, router_W)


def _thresh_body(k, keys_ref, t_ref):
    keys = keys_ref[...]
    t = jnp.full((N_EXPERTS, 1), _INT_MIN, jnp.int32)
    for b in range(31, -1, -1):
        cand = t + np.array(1 << b, dtype=np.uint32).view(np.int32)
        cnt = jnp.sum((keys >= cand).astype(jnp.int32), axis=1, keepdims=True)
        t = jnp.where(cnt >= k, cand, t)
    t_ref[...] = jnp.broadcast_to(t, (N_EXPERTS, 128))


def _threshold(keys, k):
    n = keys.shape[1]
    return pl.pallas_call(
        functools.partial(_thresh_body, k),
        in_specs=[pl.BlockSpec((N_EXPERTS, n), lambda: (0, 0))],
        out_specs=pl.BlockSpec((N_EXPERTS, 128), lambda: (0, 0)),
        out_shape=jax.ShapeDtypeStruct((N_EXPERTS, 128), jnp.int32),
    )(keys)


def _make_sc_compact(n):
    n_vregs = n // 16
    mesh = plsc.VectorSubcoreMesh(core_axis_name="c", subcore_axis_name="s")

    @functools.partial(
        pl.kernel,
        mesh=mesh,
        compiler_params=pltpu.CompilerParams(needs_layout_passes=False),
        out_type=(
            jax.ShapeDtypeStruct((N_EXPERTS * CAP,), jnp.int32),
            jax.ShapeDtypeStruct((N_EXPERTS * CAP,), jnp.int32),
        ),
        scratch_types=[
            pltpu.VMEM((n,), jnp.int32),
            pltpu.VMEM((16,), jnp.int32),
            pltpu.VMEM((CAP,), jnp.int32),
            pltpu.VMEM((CAP,), jnp.int32),
        ],
    )
    def compact_k(keys_hbm, t_hbm, keysc_hbm, idxc_hbm, keys_v, t_v, kc_v, ic_v):
        wid = lax.axis_index("s") * _NC + lax.axis_index("c")
        for t in range(_EPW):
            e = wid * _EPW + t
            pltpu.sync_copy(keys_hbm.at[pl.ds(e * n, n)], keys_v)
            pltpu.sync_copy(t_hbm.at[pl.ds(e * 16, 16)], t_v)
            tvec = t_v[...]

            def fill(j, carry):
                kc_v[pl.ds(j * 16, 16)] = jnp.full((16,), _INT_MIN, jnp.int32)
                ic_v[pl.ds(j * 16, 16)] = jnp.full((16,), 0x7FFFFFFF, jnp.int32)
                return carry

            lax.fori_loop(0, CAP // 16, fill, 0)

            def step(i, cnt):
                kv = keys_v[pl.ds(i * 16, 16)]
                m = (kv >= tvec) & (cnt < CAP - 16)
                iv = lax.iota(jnp.int32, 16) + i * 16
                cs = plsc.cumsum(m.astype(jnp.int32))
                dest = cnt + cs - 1
                plsc.store_scatter(kc_v, [dest], kv, mask=m)
                plsc.store_scatter(ic_v, [dest], iv, mask=m)
                return cnt + jnp.max(cs)

            lax.fori_loop(0, n_vregs, step, jnp.int32(0))
            pltpu.sync_copy(kc_v, keysc_hbm.at[pl.ds(e * CAP, CAP)])
            pltpu.sync_copy(ic_v, idxc_hbm.at[pl.ds(e * CAP, CAP)])

    return compact_k


def _rank_body(keys_ref, rank_ref):
    keys = keys_ref[0, 0]
    kc = keys.reshape(CAP, 1)
    kr = keys.reshape(1, CAP)
    row = lax.broadcasted_iota(jnp.int32, (CAP, CAP), 0)
    col = lax.broadcasted_iota(jnp.int32, (CAP, CAP), 1)
    a = (kr > kc) | ((kr == kc) & (col < row))
    rank_ref[0, 0] = jnp.sum(a.astype(jnp.int32), axis=1)


def _ranks(keysc):
    return pl.pallas_call(
        _rank_body,
        grid=(N_EXPERTS,),
        in_specs=[pl.BlockSpec((1, 1, CAP), lambda i: (i, 0, 0))],
        out_specs=pl.BlockSpec((1, 1, CAP), lambda i: (i, 0, 0)),
        out_shape=jax.ShapeDtypeStruct((N_EXPERTS, 1, CAP), jnp.int32),
    )(keysc.reshape(N_EXPERTS, 1, CAP))


def _make_sc_place_gather(n, k, cw):
    ch = 64
    nch = k // ch
    mesh = plsc.VectorSubcoreMesh(core_axis_name="c", subcore_axis_name="s")

    @functools.partial(
        pl.kernel,
        mesh=mesh,
        compiler_params=pltpu.CompilerParams(needs_layout_passes=False),
        out_type=(
            jax.ShapeDtypeStruct((N_EXPERTS * k,), jnp.int32),
            jax.ShapeDtypeStruct((N_EXPERTS * k,), jnp.float32),
            jax.ShapeDtypeStruct((_NW * n,), jnp.float32),
            jax.ShapeDtypeStruct((N_EXPERTS * k, cw), jnp.int32),
        ),
        scratch_types=[
            pltpu.VMEM((CAP,), jnp.int32),
            pltpu.VMEM((CAP,), jnp.int32),
            pltpu.VMEM((k,), jnp.int32),
            pltpu.VMEM((n,), jnp.float32),
            pltpu.VMEM((k,), jnp.float32),
            pltpu.VMEM((n,), jnp.float32),
            pltpu.VMEM((64, cw), jnp.int32),
            pltpu.VMEM((64, cw), jnp.int32),
            pltpu.SemaphoreType.DMA,
            pltpu.SemaphoreType.DMA,
            pltpu.SemaphoreType.DMA,
        ],
    )
    def place_k(idxc_hbm, rank_hbm, allwt_hbm, zi_hbm, zf_hbm, xb_hbm,
                lidx_hbm, wflat_hbm, fpart_hbm, xg_hbm,
                ic_v, rk_v, tk_v, aw_v, w_v, hist_v, rb0, rb1, gsem, os0, os1):
        wid = lax.axis_index("s") * _NC + lax.axis_index("c")
        pltpu.sync_copy(zf_hbm, hist_v)
        ones = jnp.full((16,), 1.0, jnp.float32)
        rbufs = (rb0, rb1)
        osems = (os0, os1)
        for t in range(_EPW):
            e = wid * _EPW + t
            pltpu.sync_copy(idxc_hbm.at[pl.ds(e * CAP, CAP)], ic_v)
            pltpu.sync_copy(rank_hbm.at[pl.ds(e * CAP, CAP)], rk_v)
            pltpu.sync_copy(zi_hbm.at[pl.ds(0, k)], tk_v)

            def place(j, carry):
                r = rk_v[pl.ds(j * 16, 16)]
                iv = ic_v[pl.ds(j * 16, 16)]
                m = r < k
                plsc.store_scatter(tk_v, [r], iv, mask=m)
                return carry

            lax.fori_loop(0, CAP // 16, place, 0)
            pltpu.sync_copy(tk_v, lidx_hbm.at[pl.ds(e * k, k)])
            pltpu.sync_copy(allwt_hbm.at[pl.ds(e * n, n)], aw_v)

            def wgather(j, carry):
                tok = tk_v[pl.ds(j * 16, 16)]
                w_v[pl.ds(j * 16, 16)] = plsc.load_gather(aw_v, [tok])
                plsc.addupdate_scatter(hist_v, [tok], ones)
                return carry

            lax.fori_loop(0, k // 16, wgather, 0)
            pltpu.sync_copy(w_v, wflat_hbm.at[pl.ds(e * k, k)])

            ch = 64
            out_cps = [None, None]
            for i in range(k // ch):
                b = i & 1
                if out_cps[b] is not None:
                    out_cps[b].wait()
                pltpu.async_copy(xb_hbm.at[tk_v.at[pl.ds(i * ch, ch)]],
                                 rbufs[b], gsem).wait()
                out_cps[b] = pltpu.async_copy(
                    rbufs[b], xg_hbm.at[pl.ds(e * k + i * ch, ch)], osems[b])
            out_cps[0].wait()
            out_cps[1].wait()
        pltpu.sync_copy(hist_v, fpart_hbm.at[pl.ds(wid * n, n)])

    return place_k


def _experts_body(xg_ref, w1_ref, w2_ref, fp_ref, out_ref, fo_ref):
    u = lax.bitcast_convert_type(xg_ref[...], jnp.uint32)
    h1 = lax.bitcast_convert_type((u >> 16).astype(jnp.uint16), jnp.bfloat16)
    h2 = lax.bitcast_convert_type(
        (u & jnp.uint32(0xFFFF)).astype(jnp.uint16), jnp.bfloat16)
    x = jnp.concatenate([h1, h2], axis=1)
    h = lax.dot_general(x, w1_ref[0].astype(jnp.bfloat16),
                        (((1,), (1,)), ((), ())),
                        preferred_element_type=jnp.float32)
    h = jax.nn.gelu(h)
    out_ref[...] = lax.dot_general(h, w2_ref[0], (((1,), (1,)), ((), ())),
                                   preferred_element_type=jnp.float32)
    @pl.when((pl.program_id(0) == 0) & (pl.program_id(1) == 0))
    def _():
        fo_ref[...] = jnp.sum(fp_ref[...], axis=0, keepdims=True)


def _experts(xg, expert_W1, expert_W2, fpart, k):
    e, d, c = expert_W1.shape
    n = fpart.shape[1]
    rb = 512
    nrb = k // rb
    return pl.pallas_call(
        _experts_body,
        grid=(e, nrb),
        in_specs=[
            pl.BlockSpec((rb, c // 2), lambda i, j: (i * nrb + j, 0)),
            pl.BlockSpec((1, d, c), lambda i, j: (i, 0, 0)),
            pl.BlockSpec((1, c, d), lambda i, j: (i, 0, 0)),
            pl.BlockSpec((_NW, n), lambda i, j: (0, 0)),
        ],
        out_specs=[
            pl.BlockSpec((rb, c), lambda i, j: (i * nrb + j, 0)),
            pl.BlockSpec((1, n), lambda i, j: (0, 0)),
        ],
        out_shape=[
            jax.ShapeDtypeStruct((e * k, c), jnp.float32),
            jax.ShapeDtypeStruct((1, n), jnp.float32),
        ],
        compiler_params=pltpu.CompilerParams(
            dimension_semantics=("arbitrary", "arbitrary")),
    )(xg, expert_W1, expert_W2, fpart)


def kernel(x, router_W, expert_W1, expert_W2):
    B, T, C = x.shape
    n_tokens = B * T
    k = n_tokens // EXPANSION
    x_flat = x.reshape(-1, C)

    allwt, keys, xbf, t_bcast = _router(x_flat, router_W, k)
    t_sc = t_bcast[:, :16].reshape(-1)
    keysc, idxc = _make_sc_compact(n_tokens)(keys.reshape(-1), t_sc)
    ranks = _ranks(keysc.reshape(N_EXPERTS, CAP))

    zi = jnp.zeros((n_tokens,), jnp.int32)
    zf = jnp.zeros((n_tokens,), jnp.float32)
    local_indices, weights_flat, fpart, xg = _make_sc_place_gather(
        n_tokens, k, C // 2)(idxc, ranks.reshape(-1), allwt.reshape(-1),
                             zi, zf, xbf)
    h_flat, fo = _experts(xg, expert_W1, expert_W2,
                          fpart.reshape(_NW, n_tokens), k)
    return h_flat, local_indices, weights_flat, fo.reshape(n_tokens)


# speedup vs baseline: 1.5234x; 1.1221x over previous
import functools

import numpy as np
import jax
import jax.numpy as jnp
from jax import lax
from jax.experimental import pallas as pl
from jax.experimental.pallas import tpu as pltpu
from jax.experimental.pallas import tpu_sc as plsc

N_EXPERTS = 64
EXPANSION = 8
CAP = 1152

try:
    _SC_INFO = plsc.get_sparse_core_info()
    _NC, _NS = _SC_INFO.num_cores, _SC_INFO.num_subcores
except Exception:
    _NC, _NS = 2, 16
_NW = _NC * _NS
_EPW = N_EXPERTS // _NW

_INT_MIN = np.int32(-2147483648)


def _router_body(nsteps, k, x_ref, rw_ref, allwt_ref, keys_ref, xb_ref,
                 t_ref, keys_acc):
    lgt = lax.dot_general(rw_ref[...].astype(jnp.bfloat16),
                          x_ref[...].astype(jnp.bfloat16),
                          (((1,), (1,)), ((), ())),
                          preferred_element_type=jnp.float32)
    m = jnp.max(lgt, axis=0, keepdims=True)
    e = jnp.exp(lgt - m)
    allwt_ref[...] = e / jnp.sum(e, axis=0, keepdims=True)
    u = lax.bitcast_convert_type(lgt, jnp.uint32)
    neg = u >= jnp.uint32(0x80000000)
    up = jnp.where(neg, ~u, u | jnp.uint32(0x80000000))
    ikeys = lax.bitcast_convert_type(up ^ jnp.uint32(0x80000000), jnp.int32)
    keys_ref[...] = ikeys
    blk = ikeys.shape[1]
    keys_acc[:, pl.ds(pl.program_id(0) * blk, blk)] = ikeys

    @pl.when(pl.program_id(0) == nsteps - 1)
    def _():
        keys = keys_acc[...]
        t = jnp.full((N_EXPERTS, 1), _INT_MIN, jnp.int32)
        for b in range(31, -1, -1):
            cand = t + np.array(1 << b, dtype=np.uint32).view(np.int32)
            cnt = jnp.sum((keys >= cand).astype(jnp.int32), axis=1,
                          keepdims=True)
            t = jnp.where(cnt >= k, cand, t)
        t_ref[...] = jnp.broadcast_to(t, (N_EXPERTS, 128))

    xb = x_ref[...].astype(jnp.bfloat16)
    c = xb.shape[1]
    a = lax.bitcast_convert_type(xb[:, : c // 2], jnp.uint16).astype(jnp.uint32)
    b = lax.bitcast_convert_type(xb[:, c // 2 :], jnp.uint16).astype(jnp.uint32)
    xb_ref[...] = lax.bitcast_convert_type((a << 16) | b, jnp.int32)


def _router(x_flat, router_W, k):
    n, c = x_flat.shape
    blk = 1024
    return pl.pallas_call(
        functools.partial(_router_body, n // blk, k),
        grid=(n // blk,),
        in_specs=[
            pl.BlockSpec((blk, c), lambda i: (i, 0)),
            pl.BlockSpec((N_EXPERTS, c), lambda i: (0, 0)),
        ],
        out_specs=[
            pl.BlockSpec((N_EXPERTS, blk), lambda i: (0, i)),
            pl.BlockSpec((N_EXPERTS, blk), lambda i: (0, i)),
            pl.BlockSpec((blk, c // 2), lambda i: (i, 0)),
            pl.BlockSpec((N_EXPERTS, 128), lambda i: (0, 0)),
        ],
        out_shape=[
            jax.ShapeDtypeStruct((N_EXPERTS, n), jnp.float32),
            jax.ShapeDtypeStruct((N_EXPERTS, n), jnp.int32),
            jax.ShapeDtypeStruct((n, c // 2), jnp.int32),
            jax.ShapeDtypeStruct((N_EXPERTS, 128), jnp.int32),
        ],
        scratch_shapes=[pltpu.VMEM((N_EXPERTS, n), jnp.int32)],
    )(x_flat, router_W)


def _thresh_body(k, keys_ref, t_ref):
    keys = keys_ref[...]
    t = jnp.full((N_EXPERTS, 1), _INT_MIN, jnp.int32)
    for b in range(31, -1, -1):
        cand = t + np.array(1 << b, dtype=np.uint32).view(np.int32)
        cnt = jnp.sum((keys >= cand).astype(jnp.int32), axis=1, keepdims=True)
        t = jnp.where(cnt >= k, cand, t)
    t_ref[...] = jnp.broadcast_to(t, (N_EXPERTS, 128))


def _threshold(keys, k):
    n = keys.shape[1]
    return pl.pallas_call(
        functools.partial(_thresh_body, k),
        in_specs=[pl.BlockSpec((N_EXPERTS, n), lambda: (0, 0))],
        out_specs=pl.BlockSpec((N_EXPERTS, 128), lambda: (0, 0)),
        out_shape=jax.ShapeDtypeStruct((N_EXPERTS, 128), jnp.int32),
    )(keys)


def _make_sc_compact(n):
    n_vregs = n // 16
    mesh = plsc.VectorSubcoreMesh(core_axis_name="c", subcore_axis_name="s")

    @functools.partial(
        pl.kernel,
        mesh=mesh,
        compiler_params=pltpu.CompilerParams(needs_layout_passes=False),
        out_type=(
            jax.ShapeDtypeStruct((N_EXPERTS * CAP,), jnp.int32),
            jax.ShapeDtypeStruct((N_EXPERTS * CAP,), jnp.int32),
        ),
        scratch_types=[
            pltpu.VMEM((n,), jnp.int32),
            pltpu.VMEM((16,), jnp.int32),
            pltpu.VMEM((CAP,), jnp.int32),
            pltpu.VMEM((CAP,), jnp.int32),
        ],
    )
    def compact_k(keys_hbm, t_hbm, keysc_hbm, idxc_hbm, keys_v, t_v, kc_v, ic_v):
        wid = lax.axis_index("s") * _NC + lax.axis_index("c")
        for t in range(_EPW):
            e = wid * _EPW + t
            pltpu.sync_copy(keys_hbm.at[pl.ds(e * n, n)], keys_v)
            pltpu.sync_copy(t_hbm.at[pl.ds(e * 16, 16)], t_v)
            tvec = t_v[...]

            def fill(j, carry):
                kc_v[pl.ds(j * 16, 16)] = jnp.full((16,), _INT_MIN, jnp.int32)
                ic_v[pl.ds(j * 16, 16)] = jnp.full((16,), 0x7FFFFFFF, jnp.int32)
                return carry

            lax.fori_loop(0, CAP // 16, fill, 0)

            def step(i, cnt):
                kv = keys_v[pl.ds(i * 16, 16)]
                m = (kv >= tvec) & (cnt < CAP - 16)
                iv = lax.iota(jnp.int32, 16) + i * 16
                cs = plsc.cumsum(m.astype(jnp.int32))
                dest = cnt + cs - 1
                plsc.store_scatter(kc_v, [dest], kv, mask=m)
                plsc.store_scatter(ic_v, [dest], iv, mask=m)
                return cnt + jnp.max(cs)

            lax.fori_loop(0, n_vregs, step, jnp.int32(0))
            pltpu.sync_copy(kc_v, keysc_hbm.at[pl.ds(e * CAP, CAP)])
            pltpu.sync_copy(ic_v, idxc_hbm.at[pl.ds(e * CAP, CAP)])

    return compact_k


def _rank_body(keys_ref, rank_ref):
    keys = keys_ref[0, 0]
    kc = keys.reshape(CAP, 1)
    kr = keys.reshape(1, CAP)
    row = lax.broadcasted_iota(jnp.int32, (CAP, CAP), 0)
    col = lax.broadcasted_iota(jnp.int32, (CAP, CAP), 1)
    a = (kr > kc) | ((kr == kc) & (col < row))
    rank_ref[0, 0] = jnp.sum(a.astype(jnp.int32), axis=1)


def _ranks(keysc):
    return pl.pallas_call(
        _rank_body,
        grid=(N_EXPERTS,),
        in_specs=[pl.BlockSpec((1, 1, CAP), lambda i: (i, 0, 0))],
        out_specs=pl.BlockSpec((1, 1, CAP), lambda i: (i, 0, 0)),
        out_shape=jax.ShapeDtypeStruct((N_EXPERTS, 1, CAP), jnp.int32),
    )(keysc.reshape(N_EXPERTS, 1, CAP))


def _make_sc_place_gather(n, k, cw):
    ch = 64
    nch = k // ch
    mesh = plsc.VectorSubcoreMesh(core_axis_name="c", subcore_axis_name="s")

    @functools.partial(
        pl.kernel,
        mesh=mesh,
        compiler_params=pltpu.CompilerParams(needs_layout_passes=False),
        out_type=(
            jax.ShapeDtypeStruct((N_EXPERTS * k,), jnp.int32),
            jax.ShapeDtypeStruct((N_EXPERTS * k,), jnp.float32),
            jax.ShapeDtypeStruct((_NW * n,), jnp.float32),
            jax.ShapeDtypeStruct((N_EXPERTS * k, cw), jnp.int32),
        ),
        scratch_types=[
            pltpu.VMEM((CAP,), jnp.int32),
            pltpu.VMEM((CAP,), jnp.int32),
            pltpu.VMEM((k,), jnp.int32),
            pltpu.VMEM((n,), jnp.float32),
            pltpu.VMEM((k,), jnp.float32),
            pltpu.VMEM((n,), jnp.float32),
            pltpu.VMEM((64, cw), jnp.int32),
            pltpu.VMEM((64, cw), jnp.int32),
            pltpu.SemaphoreType.DMA,
            pltpu.SemaphoreType.DMA,
            pltpu.SemaphoreType.DMA,
        ],
    )
    def place_k(idxc_hbm, rank_hbm, allwt_hbm, zi_hbm, zf_hbm, xb_hbm,
                lidx_hbm, wflat_hbm, fpart_hbm, xg_hbm,
                ic_v, rk_v, tk_v, aw_v, w_v, hist_v, rb0, rb1, gsem, os0, os1):
        wid = lax.axis_index("s") * _NC + lax.axis_index("c")
        pltpu.sync_copy(zf_hbm, hist_v)
        ones = jnp.full((16,), 1.0, jnp.float32)
        rbufs = (rb0, rb1)
        osems = (os0, os1)
        for t in range(_EPW):
            e = wid * _EPW + t
            pltpu.sync_copy(idxc_hbm.at[pl.ds(e * CAP, CAP)], ic_v)
            pltpu.sync_copy(rank_hbm.at[pl.ds(e * CAP, CAP)], rk_v)
            pltpu.sync_copy(zi_hbm.at[pl.ds(0, k)], tk_v)

            def place(j, carry):
                r = rk_v[pl.ds(j * 16, 16)]
                iv = ic_v[pl.ds(j * 16, 16)]
                m = r < k
                plsc.store_scatter(tk_v, [r], iv, mask=m)
                return carry

            lax.fori_loop(0, CAP // 16, place, 0)
            pltpu.sync_copy(tk_v, lidx_hbm.at[pl.ds(e * k, k)])
            pltpu.sync_copy(allwt_hbm.at[pl.ds(e * n, n)], aw_v)

            def wgather(j, carry):
                tok = tk_v[pl.ds(j * 16, 16)]
                w_v[pl.ds(j * 16, 16)] = plsc.load_gather(aw_v, [tok])
                plsc.addupdate_scatter(hist_v, [tok], ones)
                return carry

            lax.fori_loop(0, k // 16, wgather, 0)
            pltpu.sync_copy(w_v, wflat_hbm.at[pl.ds(e * k, k)])

            ch = 64
            out_cps = [None, None]
            for i in range(k // ch):
                b = i & 1
                if out_cps[b] is not None:
                    out_cps[b].wait()
                pltpu.async_copy(xb_hbm.at[tk_v.at[pl.ds(i * ch, ch)]],
                                 rbufs[b], gsem).wait()
                out_cps[b] = pltpu.async_copy(
                    rbufs[b], xg_hbm.at[pl.ds(e * k + i * ch, ch)], osems[b])
            out_cps[0].wait()
            out_cps[1].wait()
        pltpu.sync_copy(hist_v, fpart_hbm.at[pl.ds(wid * n, n)])

    return place_k


def _experts_body(xg_ref, w1_ref, w2_ref, fp_ref, out_ref, fo_ref):
    u = lax.bitcast_convert_type(xg_ref[...], jnp.uint32)
    h1 = lax.bitcast_convert_type((u >> 16).astype(jnp.uint16), jnp.bfloat16)
    h2 = lax.bitcast_convert_type(
        (u & jnp.uint32(0xFFFF)).astype(jnp.uint16), jnp.bfloat16)
    x = jnp.concatenate([h1, h2], axis=1)
    h = lax.dot_general(x, w1_ref[0].astype(jnp.bfloat16),
                        (((1,), (1,)), ((), ())),
                        preferred_element_type=jnp.float32)
    h = jax.nn.gelu(h)
    out_ref[...] = lax.dot_general(h, w2_ref[0], (((1,), (1,)), ((), ())),
                                   preferred_element_type=jnp.float32)
    @pl.when((pl.program_id(0) == 0) & (pl.program_id(1) == 0))
    def _():
        fo_ref[...] = jnp.sum(fp_ref[...], axis=0, keepdims=True)


def _experts(xg, expert_W1, expert_W2, fpart, k):
    e, d, c = expert_W1.shape
    n = fpart.shape[1]
    rb = 1024
    nrb = k // rb
    return pl.pallas_call(
        _experts_body,
        grid=(e, nrb),
        in_specs=[
            pl.BlockSpec((rb, c // 2), lambda i, j: (i * nrb + j, 0)),
            pl.BlockSpec((1, d, c), lambda i, j: (i, 0, 0)),
            pl.BlockSpec((1, c, d), lambda i, j: (i, 0, 0)),
            pl.BlockSpec((_NW, n), lambda i, j: (0, 0)),
        ],
        out_specs=[
            pl.BlockSpec((rb, c), lambda i, j: (i * nrb + j, 0)),
            pl.BlockSpec((1, n), lambda i, j: (0, 0)),
        ],
        out_shape=[
            jax.ShapeDtypeStruct((e * k, c), jnp.float32),
            jax.ShapeDtypeStruct((1, n), jnp.float32),
        ],
        compiler_params=pltpu.CompilerParams(
            dimension_semantics=("arbitrary", "arbitrary")),
    )(xg, expert_W1, expert_W2, fpart)


def kernel(x, router_W, expert_W1, expert_W2):
    B, T, C = x.shape
    n_tokens = B * T
    k = n_tokens // EXPANSION
    x_flat = x.reshape(-1, C)

    allwt, keys, xbf, t_bcast = _router(x_flat, router_W, k)
    t_sc = t_bcast[:, :16].reshape(-1)
    keysc, idxc = _make_sc_compact(n_tokens)(keys.reshape(-1), t_sc)
    ranks = _ranks(keysc.reshape(N_EXPERTS, CAP))

    zi = jnp.zeros((n_tokens,), jnp.int32)
    zf = jnp.zeros((n_tokens,), jnp.float32)
    local_indices, weights_flat, fpart, xg = _make_sc_place_gather(
        n_tokens, k, C // 2)(idxc, ranks.reshape(-1), allwt.reshape(-1),
                             zi, zf, xbf)
    h_flat, fo = _experts(xg, expert_W1, expert_W2,
                          fpart.reshape(_NW, n_tokens), k)
    return h_flat, local_indices, weights_flat, fo.reshape(n_tokens)
